# baseline (speedup 1.0000x reference)
import functools

import jax
import jax.numpy as jnp
from jax import lax
from jax.experimental import pallas as pl
from jax.experimental.pallas import tpu as pltpu
from jax.experimental.pallas import tpu_sc as plsc

N = 10000
NP = 10240
E = 320000
NC = 2
NS = 16
NW = NC * NS
EW = E // NW
K = 80
NCH = EW // K
RW = NP // NS
BR = 256
NB = NP // BR
F32 = jnp.float32

_mesh = plsc.VectorSubcoreMesh(core_axis_name="c", subcore_axis_name="s")


def _hi_dot(a, b):
  return lax.dot_general(a, b, (((1,), (0,)), ((), ())),
                         preferred_element_type=F32,
                         precision=lax.Precision.HIGHEST)


def _encode(x, wr, b, a_src, a_dst, w1):
  u = x.shape[1]
  h1 = w1.shape[1]
  a2 = jnp.stack([a_src, a_dst], axis=1)
  b2 = b.reshape(1, u)

  def body(x_ref, wr_ref, b_ref, a2_ref, w1_ref, h_ref, x1_ref, sd_ref):
    h = jnp.tanh(_hi_dot(x_ref[...], wr_ref[...]) + b_ref[...])
    h_ref[...] = h
    x1_ref[...] = _hi_dot(h, w1_ref[...])
    sd_ref[...] = _hi_dot(h, a2_ref[...])

  h, x1, sd = pl.pallas_call(
      body,
      grid=(NB,),
      in_specs=[
          pl.BlockSpec((BR, u), lambda i: (i, 0)),
          pl.BlockSpec((u, u), lambda i: (0, 0)),
          pl.BlockSpec((1, u), lambda i: (0, 0)),
          pl.BlockSpec((u, 2), lambda i: (0, 0)),
          pl.BlockSpec((u, h1), lambda i: (0, 0)),
      ],
      out_specs=[
          pl.BlockSpec((BR, u), lambda i: (i, 0)),
          pl.BlockSpec((BR, h1), lambda i: (i, 0)),
          pl.BlockSpec((BR, 2), lambda i: (i, 0)),
      ],
      out_shape=[
          jax.ShapeDtypeStruct((NP, u), F32),
          jax.ShapeDtypeStruct((NP, h1), F32),
          jax.ShapeDtypeStruct((NP, 2), F32),
      ],
  )(x, wr, b2, a2, w1)
  return h, x1, sd


@functools.partial(
    pl.kernel,
    out_type=[
        jax.ShapeDtypeStruct((NW, EW), F32),
        jax.ShapeDtypeStruct((NC, NP), F32),
    ],
    mesh=_mesh,
    scratch_types=[
        pltpu.VMEM((NP,), F32),
        pltpu.VMEM((NP,), F32),
        pltpu.VMEM((NCH, K), jnp.int32),
        pltpu.VMEM((NCH, K), jnp.int32),
        pltpu.VMEM((EW,), F32),
        pltpu.VMEM((RW,), F32),
        pltpu.VMEM((128,), F32),
        pltpu.VMEM_SHARED((NP,), F32),
    ],
    compiler_params=pltpu.CompilerParams(needs_layout_passes=False, use_tc_tiling_on_sc=False),
)
def _attn_kernel(s_hbm, d_hbm, src_hbm, dst_hbm, ex_out, z_out,
                 sv, dv, srcv, dstv, exbuf, zslice, tmp16, zsh):
  cid = lax.axis_index("c")
  sid = lax.axis_index("s")
  wid = sid * NC + cid
  pltpu.sync_copy(s_hbm, sv)
  pltpu.sync_copy(d_hbm, dv)
  pltpu.sync_copy(src_hbm.at[wid], srcv)
  pltpu.sync_copy(dst_hbm.at[wid], dstv)

  zero16 = jnp.zeros((16,), F32)
  for i in range(RW // 16):
    zslice[pl.ds(i * 16, 16)] = zero16
  pltpu.sync_copy(zslice, zsh.at[pl.ds(sid * RW, RW)])
  plsc.subcore_barrier()

  def maxbody(i, carry):
    ms, md = carry
    return (jnp.maximum(ms, sv[pl.ds(i * 16, 16)]),
            jnp.maximum(md, dv[pl.ds(i * 16, 16)]))

  ms, md = lax.fori_loop(0, NP // 16, maxbody,
                         (jnp.full((16,), -1e30, F32),
                          jnp.full((16,), -1e30, F32)))
  lanes = lax.iota(jnp.int32, 16)

  def lane_max(v):
    for shift in (8, 4, 2, 1):
      tmp16[pl.ds(0, 16)] = v
      v = jnp.maximum(v, plsc.load_gather(tmp16, [(lanes + shift) & 15]))
    return v

  csplat = jnp.maximum(lane_max(ms) + lane_max(md), 0.0)

  def chunk(c, carry):
    for j in range(K // 16):
      si = srcv[c, pl.ds(j * 16, 16)]
      di = dstv[c, pl.ds(j * 16, 16)]
      e = plsc.load_gather(sv, [si]) + plsc.load_gather(dv, [di])
      e = jnp.where(e >= 0.0, e, 0.2 * e)
      exbuf[pl.ds(c * K + j * 16, 16)] = jnp.exp(e - csplat)
    pltpu.sync_copy(exbuf.at[pl.ds(c * K, K)], zsh.at[dstv.at[c]], add=True)
    return carry

  lax.fori_loop(0, NCH, chunk, 0)
  pltpu.sync_copy(exbuf, ex_out.at[wid])
  plsc.subcore_barrier()
  pltpu.sync_copy(zsh.at[pl.ds(sid * RW, RW)],
                  z_out.at[cid, pl.ds(sid * RW, RW)])


def _make_msgpass(d):
  @functools.partial(
      pl.kernel,
      out_type=jax.ShapeDtypeStruct((NC, NP, d), F32),
      mesh=_mesh,
      scratch_types=[
          [pltpu.VMEM((3, K), jnp.int32)] * 4,
          [pltpu.VMEM((K, d), F32)] * 2,
          pltpu.VMEM((16, d), F32),
          pltpu.VMEM_SHARED((NP, d), F32),
          [pltpu.SemaphoreType.DMA] * 4,
          [pltpu.SemaphoreType.DMA] * 2,
          [pltpu.SemaphoreType.DMA] * 2,
      ],
      compiler_params=pltpu.CompilerParams(needs_layout_passes=False, use_tc_tiling_on_sc=False),
  )
  def msg_kernel(x_hbm, pack_hbm, acc_out,
                 idxb, rowsb, zrow, acc, isem, gsem, ssem):
    cid = lax.axis_index("c")
    sid = lax.axis_index("s")
    wid = sid * NC + cid

    zero16 = jnp.zeros((16,), F32)
    for i in range(16):
      for j in range(d // 16):
        zrow[i, pl.ds(j * 16, 16)] = zero16
    for i in range(RW // 16):
      pltpu.sync_copy(zrow, acc.at[pl.ds(sid * RW + i * 16, 16)])
    plsc.subcore_barrier()

    def start_idx(c, ib):
      pltpu.async_copy(pack_hbm.at[wid, c], idxb[ib], isem[ib])

    def wait_idx(ib):
      pltpu.make_async_copy(pack_hbm.at[wid, 0], idxb[ib], isem[ib]).wait()

    def start_gather(ib, b):
      pltpu.async_copy(x_hbm.at[idxb[ib].at[0]], rowsb[b], gsem[b])

    def wait_gather(ib, b):
      pltpu.make_async_copy(x_hbm.at[idxb[ib].at[0]], rowsb[b], gsem[b]).wait()

    def start_scatter(ib, b):
      pltpu.async_copy(rowsb[b], acc.at[idxb[ib].at[1]], ssem[b], add=True)

    def wait_scatter(ib, b):
      pltpu.make_async_copy(rowsb[b], acc.at[idxb[ib].at[1]], ssem[b]).wait()

    def scale(ib, b):
      rows = rowsb[b]

      def sbody(k0, carry):
        base = k0 * 16
        for kk in range(16):
          wbits = plsc.load_gather(
              idxb[ib], [jnp.full((16,), 2, jnp.int32),
                         jnp.full((16,), base + kk, jnp.int32)])
          w = plsc.bitcast(wbits, F32)
          for j in range(d // 16):
            rows[base + kk, pl.ds(j * 16, 16)] = (
                rows[base + kk, pl.ds(j * 16, 16)] * w)
        return carry

      lax.fori_loop(0, K // 16, sbody, 0)

    def step(c_dyn, cm, first, has_next, has_next2):
      ib, b = cm % 4, cm % 2
      wait_gather(ib, b)
      scale(ib, b)
      start_scatter(ib, b)
      if not first:
        wait_scatter((ib - 1) % 4, b ^ 1)
      if has_next:
        wait_idx((ib + 1) % 4)
        start_gather((ib + 1) % 4, b ^ 1)
      if has_next2:
        start_idx(c_dyn + 2, (ib + 2) % 4)

    start_idx(0, 0)
    start_idx(1, 1)
    wait_idx(0)
    start_gather(0, 0)
    step(0, 0, True, True, True)
    step(1, 1, False, True, True)
    step(2, 2, False, True, True)

    def body(t, carry):
      c0 = 4 * t + 3
      step(c0, 3, False, True, True)
      step(c0 + 1, 0, False, True, True)
      step(c0 + 2, 1, False, True, True)
      step(c0 + 3, 2, False, True, True)
      return carry

    lax.fori_loop(0, (NCH - 5) // 4, body, 0)
    step(NCH - 2, (NCH - 2) % 4, False, True, False)
    step(NCH - 1, (NCH - 1) % 4, False, False, False)
    wait_scatter((NCH - 1) % 4, (NCH - 1) % 2)

    plsc.subcore_barrier()
    pltpu.sync_copy(acc.at[pl.ds(sid * RW, RW)],
                    acc_out.at[cid, pl.ds(sid * RW, RW)])

  return msg_kernel


_msgpass_128 = _make_msgpass(128)
_msgpass_64 = _make_msgpass(64)


def _layer1(acc, z3, w2):
  h1, h2 = w2.shape

  def body(acc_ref, z_ref, w2_ref, x2_ref):
    den = z_ref[0] + z_ref[1] + 1e-16
    g = jnp.maximum((acc_ref[0] + acc_ref[1]) / den, 0.0)
    x2_ref[...] = _hi_dot(g, w2_ref[...])

  return pl.pallas_call(
      body,
      grid=(NB,),
      in_specs=[
          pl.BlockSpec((NC, BR, h1), lambda i: (0, i, 0)),
          pl.BlockSpec((NC, BR, 1), lambda i: (0, i, 0)),
          pl.BlockSpec((h1, h2), lambda i: (0, 0)),
      ],
      out_specs=pl.BlockSpec((BR, h2), lambda i: (i, 0)),
      out_shape=jax.ShapeDtypeStruct((NP, h2), F32),
  )(acc, z3, w2)


def _final(acc2_t, acc2_a, z3_t, z3_a, h_t, h_a, wc1, bc1, wc2, bc2):
  h2 = acc2_t.shape[2]
  ut = h_t.shape[1]
  ua = h_a.shape[1]
  hc = wc1.shape[1]
  tags = wc2.shape[1]
  w_g2t = wc1[0:h2]
  w_g2a = wc1[h2:2 * h2]
  w_ha = wc1[2 * h2:2 * h2 + ua]
  w_ht = wc1[2 * h2 + ua:]
  bc1r = bc1.reshape(1, hc)
  bc2r = bc2.reshape(1, tags)

  def body(a2t_ref, a2a_ref, zt_ref, za_ref, ht_ref, ha_ref,
           wg2t_ref, wg2a_ref, wha_ref, wht_ref, b1_ref, wc2_ref, b2_ref,
           out_ref):
    g2t = (a2t_ref[0] + a2t_ref[1]) / (zt_ref[0] + zt_ref[1] + 1e-16)
    g2a = (a2a_ref[0] + a2a_ref[1]) / (za_ref[0] + za_ref[1] + 1e-16)
    hid = (_hi_dot(g2t, wg2t_ref[...]) + _hi_dot(g2a, wg2a_ref[...])
           + _hi_dot(ha_ref[...], wha_ref[...])
           + _hi_dot(ht_ref[...], wht_ref[...]) + b1_ref[...])
    hid = jnp.maximum(hid, 0.0)
    lg = _hi_dot(hid, wc2_ref[...]) + b2_ref[...]
    m = jnp.max(lg, axis=1, keepdims=True)
    p = lg - m
    out_ref[...] = p - jnp.log(jnp.sum(jnp.exp(p), axis=1, keepdims=True))

  return pl.pallas_call(
      body,
      grid=(NB,),
      in_specs=[
          pl.BlockSpec((NC, BR, h2), lambda i: (0, i, 0)),
          pl.BlockSpec((NC, BR, h2), lambda i: (0, i, 0)),
          pl.BlockSpec((NC, BR, 1), lambda i: (0, i, 0)),
          pl.BlockSpec((NC, BR, 1), lambda i: (0, i, 0)),
          pl.BlockSpec((BR, ut), lambda i: (i, 0)),
          pl.BlockSpec((BR, ua), lambda i: (i, 0)),
          pl.BlockSpec((h2, hc), lambda i: (0, 0)),
          pl.BlockSpec((h2, hc), lambda i: (0, 0)),
          pl.BlockSpec((ua, hc), lambda i: (0, 0)),
          pl.BlockSpec((ut, hc), lambda i: (0, 0)),
          pl.BlockSpec((1, hc), lambda i: (0, 0)),
          pl.BlockSpec((hc, tags), lambda i: (0, 0)),
          pl.BlockSpec((1, tags), lambda i: (0, 0)),
      ],
      out_specs=pl.BlockSpec((BR, tags), lambda i: (i, 0)),
      out_shape=jax.ShapeDtypeStruct((N, tags), F32),
  )(acc2_t, acc2_a, z3_t, z3_a, h_t, h_a,
    w_g2t, w_g2a, w_ha, w_ht, bc1r, wc2, bc2r)


def kernel(train_text, train_audio, edge_index, W_rnn_t, b_rnn_t, W_rnn_a,
           b_rnn_a, a_src_t, a_dst_t, a_src_a, a_dst_a, W1_t, W2_t, W1_a,
           W2_a, Wc1, bc1, Wc2, bc2):
  src3 = edge_index[0].reshape(NW, NCH, K)
  dst3 = edge_index[1].reshape(NW, NCH, K)
  xt = jnp.pad(train_text, ((0, NP - N), (0, 0)))
  xa = jnp.pad(train_audio, ((0, NP - N), (0, 0)))

  h_t, x1_t, sd_t = _encode(xt, W_rnn_t, b_rnn_t, a_src_t, a_dst_t, W1_t)
  h_a, x1_a, sd_a = _encode(xa, W_rnn_a, b_rnn_a, a_src_a, a_dst_a, W1_a)

  ex_t, z_t = _attn_kernel(sd_t[:, 0], sd_t[:, 1], src3, dst3)
  ex_a, z_a = _attn_kernel(sd_a[:, 0], sd_a[:, 1], src3, dst3)
  z3_t = z_t.reshape(NC, NP, 1)
  z3_a = z_a.reshape(NC, NP, 1)

  def packed(ex):
    exbits = lax.bitcast_convert_type(ex.reshape(NW, NCH, K), jnp.int32)
    return jnp.stack([src3, dst3, exbits], axis=2)

  pack_t = packed(ex_t)
  pack_a = packed(ex_a)

  acc1_t = _msgpass_128(x1_t, pack_t)
  acc1_a = _msgpass_128(x1_a, pack_a)

  x2_t = _layer1(acc1_t, z3_t, W2_t)
  x2_a = _layer1(acc1_a, z3_a, W2_a)

  acc2_t = _msgpass_64(x2_t, pack_t)
  acc2_a = _msgpass_64(x2_a, pack_a)

  return _final(acc2_t, acc2_a, z3_t, z3_a, h_t, h_a, Wc1, bc1, Wc2, bc2)


# speedup vs baseline: 1.2025x; 1.2025x over previous
import functools

import jax
import jax.numpy as jnp
from jax import lax
from jax.experimental import pallas as pl
from jax.experimental.pallas import tpu as pltpu
from jax.experimental.pallas import tpu_sc as plsc

N = 10000
NP = 10240
E = 320000
NC = 2
NS = 16
NW = NC * NS
EW = E // NW
K = 80
NCH = EW // K
RW = NP // NS
BR = 256
NB = NP // BR
F32 = jnp.float32

_mesh = plsc.VectorSubcoreMesh(core_axis_name="c", subcore_axis_name="s")


def _hi_dot(a, b):
  return lax.dot_general(a, b, (((1,), (0,)), ((), ())),
                         preferred_element_type=F32,
                         precision=lax.Precision.HIGHEST)


def _encode(x, wr, b, a_src, a_dst, w1):
  u = x.shape[1]
  h1 = w1.shape[1]
  a2 = jnp.stack([a_src, a_dst], axis=1)
  b2 = b.reshape(1, u)

  def body(x_ref, wr_ref, b_ref, a2_ref, w1_ref, h_ref, x1_ref, sd_ref):
    h = jnp.tanh(_hi_dot(x_ref[...], wr_ref[...]) + b_ref[...])
    h_ref[...] = h
    x1_ref[...] = _hi_dot(h, w1_ref[...])
    sd_ref[...] = _hi_dot(h, a2_ref[...])

  h, x1, sd = pl.pallas_call(
      body,
      grid=(NB,),
      in_specs=[
          pl.BlockSpec((BR, u), lambda i: (i, 0)),
          pl.BlockSpec((u, u), lambda i: (0, 0)),
          pl.BlockSpec((1, u), lambda i: (0, 0)),
          pl.BlockSpec((u, 2), lambda i: (0, 0)),
          pl.BlockSpec((u, h1), lambda i: (0, 0)),
      ],
      out_specs=[
          pl.BlockSpec((BR, u), lambda i: (i, 0)),
          pl.BlockSpec((BR, h1), lambda i: (i, 0)),
          pl.BlockSpec((BR, 2), lambda i: (i, 0)),
      ],
      out_shape=[
          jax.ShapeDtypeStruct((NP, u), F32),
          jax.ShapeDtypeStruct((NP, h1), F32),
          jax.ShapeDtypeStruct((NP, 2), F32),
      ],
  )(x, wr, b2, a2, w1)
  return h, x1, sd


@functools.partial(
    pl.kernel,
    out_type=[
        jax.ShapeDtypeStruct((NW, EW), F32),
        jax.ShapeDtypeStruct((NC, NP), F32),
    ],
    mesh=_mesh,
    scratch_types=[
        pltpu.VMEM((NP,), F32),
        pltpu.VMEM((NP,), F32),
        pltpu.VMEM((NCH, K), jnp.int32),
        pltpu.VMEM((NCH, K), jnp.int32),
        pltpu.VMEM((EW,), F32),
        pltpu.VMEM((RW,), F32),
        pltpu.VMEM((128,), F32),
        pltpu.VMEM_SHARED((NP,), F32),
    ],
    compiler_params=pltpu.CompilerParams(needs_layout_passes=False, use_tc_tiling_on_sc=False),
)
def _attn_kernel(s_hbm, d_hbm, src_hbm, dst_hbm, ex_out, z_out,
                 sv, dv, srcv, dstv, exbuf, zslice, tmp16, zsh):
  cid = lax.axis_index("c")
  sid = lax.axis_index("s")
  wid = sid * NC + cid
  pltpu.sync_copy(s_hbm, sv)
  pltpu.sync_copy(d_hbm, dv)
  pltpu.sync_copy(src_hbm.at[wid], srcv)
  pltpu.sync_copy(dst_hbm.at[wid], dstv)

  zero16 = jnp.zeros((16,), F32)
  for i in range(RW // 16):
    zslice[pl.ds(i * 16, 16)] = zero16
  pltpu.sync_copy(zslice, zsh.at[pl.ds(sid * RW, RW)])
  plsc.subcore_barrier()

  def maxbody(i, carry):
    ms, md = carry
    return (jnp.maximum(ms, sv[pl.ds(i * 16, 16)]),
            jnp.maximum(md, dv[pl.ds(i * 16, 16)]))

  ms, md = lax.fori_loop(0, NP // 16, maxbody,
                         (jnp.full((16,), -1e30, F32),
                          jnp.full((16,), -1e30, F32)))
  lanes = lax.iota(jnp.int32, 16)

  def lane_max(v):
    for shift in (8, 4, 2, 1):
      tmp16[pl.ds(0, 16)] = v
      v = jnp.maximum(v, plsc.load_gather(tmp16, [(lanes + shift) & 15]))
    return v

  csplat = jnp.maximum(lane_max(ms) + lane_max(md), 0.0)

  def chunk(c, carry):
    for j in range(K // 16):
      si = srcv[c, pl.ds(j * 16, 16)]
      di = dstv[c, pl.ds(j * 16, 16)]
      e = plsc.load_gather(sv, [si]) + plsc.load_gather(dv, [di])
      e = jnp.where(e >= 0.0, e, 0.2 * e)
      exbuf[pl.ds(c * K + j * 16, 16)] = jnp.exp(e - csplat)
    pltpu.sync_copy(exbuf.at[pl.ds(c * K, K)], zsh.at[dstv.at[c]], add=True)
    return carry

  lax.fori_loop(0, NCH, chunk, 0)
  pltpu.sync_copy(exbuf, ex_out.at[wid])
  plsc.subcore_barrier()
  pltpu.sync_copy(zsh.at[pl.ds(sid * RW, RW)],
                  z_out.at[cid, pl.ds(sid * RW, RW)])


def _make_msgpass(d, segs):
  @functools.partial(
      pl.kernel,
      out_type=jax.ShapeDtypeStruct((NC, NP, d), F32),
      mesh=_mesh,
      scratch_types=[
          [pltpu.VMEM((4, K), jnp.int32)] * 4,
          [pltpu.VMEM((K, d), F32)] * 2,
          pltpu.VMEM((16, d), F32),
          pltpu.VMEM_SHARED((NP, d), F32),
          [pltpu.SemaphoreType.DMA] * 4,
          [pltpu.SemaphoreType.DMA] * 2,
          [pltpu.SemaphoreType.DMA] * 2,
      ],
      compiler_params=pltpu.CompilerParams(needs_layout_passes=False, use_tc_tiling_on_sc=False),
  )
  def msg_kernel(x_hbm, pack_hbm, acc_out,
                 idxb, rowsb, zrow, acc, isem, gsem, ssem):
    cid = lax.axis_index("c")
    sid = lax.axis_index("s")
    wid = sid * NC + cid

    zero16 = jnp.zeros((16,), F32)
    for i in range(16):
      for j in range(d // 16):
        zrow[i, pl.ds(j * 16, 16)] = zero16
    for i in range(RW // 16):
      pltpu.sync_copy(zrow, acc.at[pl.ds(sid * RW + i * 16, 16)])
    plsc.subcore_barrier()

    def start_idx(c, ib):
      pltpu.async_copy(pack_hbm.at[wid, c], idxb[ib], isem[ib])

    def wait_idx(ib):
      pltpu.make_async_copy(pack_hbm.at[wid, 0], idxb[ib], isem[ib]).wait()

    def start_gather(ib, b):
      pltpu.async_copy(x_hbm.at[idxb[ib].at[0]], rowsb[b], gsem[b])

    def wait_gather(ib, b):
      pltpu.make_async_copy(x_hbm.at[idxb[ib].at[0]], rowsb[b], gsem[b]).wait()

    def start_scatter(ib, b):
      pltpu.async_copy(rowsb[b], acc.at[idxb[ib].at[1]], ssem[b], add=True)

    def wait_scatter(ib, b):
      pltpu.make_async_copy(rowsb[b], acc.at[idxb[ib].at[1]], ssem[b]).wait()

    def scale(ib, b):
      rows = rowsb[b]

      def sbody(k0, carry):
        base = k0 * 16
        for kk in range(16):
          for (prow, jlo, jhi) in segs:
            wbits = plsc.load_gather(
                idxb[ib], [jnp.full((16,), prow, jnp.int32),
                           jnp.full((16,), base + kk, jnp.int32)])
            w = plsc.bitcast(wbits, F32)
            for j in range(jlo, jhi):
              rows[base + kk, pl.ds(j * 16, 16)] = (
                  rows[base + kk, pl.ds(j * 16, 16)] * w)
        return carry

      lax.fori_loop(0, K // 16, sbody, 0)

    def step(c_dyn, cm, first, has_next, has_next2):
      ib, b = cm % 4, cm % 2
      wait_gather(ib, b)
      scale(ib, b)
      start_scatter(ib, b)
      if not first:
        wait_scatter((ib - 1) % 4, b ^ 1)
      if has_next:
        wait_idx((ib + 1) % 4)
        start_gather((ib + 1) % 4, b ^ 1)
      if has_next2:
        start_idx(c_dyn + 2, (ib + 2) % 4)

    start_idx(0, 0)
    start_idx(1, 1)
    wait_idx(0)
    start_gather(0, 0)
    step(0, 0, True, True, True)
    step(1, 1, False, True, True)
    step(2, 2, False, True, True)

    def body(t, carry):
      c0 = 4 * t + 3
      step(c0, 3, False, True, True)
      step(c0 + 1, 0, False, True, True)
      step(c0 + 2, 1, False, True, True)
      step(c0 + 3, 2, False, True, True)
      return carry

    lax.fori_loop(0, (NCH - 5) // 4, body, 0)
    step(NCH - 2, (NCH - 2) % 4, False, True, False)
    step(NCH - 1, (NCH - 1) % 4, False, False, False)
    wait_scatter((NCH - 1) % 4, (NCH - 1) % 2)

    plsc.subcore_barrier()
    pltpu.sync_copy(acc.at[pl.ds(sid * RW, RW)],
                    acc_out.at[cid, pl.ds(sid * RW, RW)])

  return msg_kernel


_msgpass_t = _make_msgpass(128, ((2, 0, 8),))
_msgpass_a = _make_msgpass(128, ((3, 0, 8),))
_msgpass_cat = _make_msgpass(128, ((2, 0, 4), (3, 4, 8)))


def _layer1(acc, z3, w2):
  h1, h2 = w2.shape

  def body(acc_ref, z_ref, w2_ref, x2_ref):
    den = z_ref[0] + z_ref[1] + 1e-16
    g = jnp.maximum((acc_ref[0] + acc_ref[1]) / den, 0.0)
    x2_ref[...] = _hi_dot(g, w2_ref[...])

  return pl.pallas_call(
      body,
      grid=(NB,),
      in_specs=[
          pl.BlockSpec((NC, BR, h1), lambda i: (0, i, 0)),
          pl.BlockSpec((NC, BR, 1), lambda i: (0, i, 0)),
          pl.BlockSpec((h1, h2), lambda i: (0, 0)),
      ],
      out_specs=pl.BlockSpec((BR, h2), lambda i: (i, 0)),
      out_shape=jax.ShapeDtypeStruct((NP, h2), F32),
  )(acc, z3, w2)


def _final(acc2_t, acc2_a, z3_t, z3_a, h_t, h_a, wc1, bc1, wc2, bc2):
  h2 = acc2_t.shape[2]
  ut = h_t.shape[1]
  ua = h_a.shape[1]
  hc = wc1.shape[1]
  tags = wc2.shape[1]
  w_g2t = wc1[0:h2]
  w_g2a = wc1[h2:2 * h2]
  w_ha = wc1[2 * h2:2 * h2 + ua]
  w_ht = wc1[2 * h2 + ua:]
  bc1r = bc1.reshape(1, hc)
  bc2r = bc2.reshape(1, tags)

  def body(a2t_ref, a2a_ref, zt_ref, za_ref, ht_ref, ha_ref,
           wg2t_ref, wg2a_ref, wha_ref, wht_ref, b1_ref, wc2_ref, b2_ref,
           out_ref):
    g2t = (a2t_ref[0] + a2t_ref[1]) / (zt_ref[0] + zt_ref[1] + 1e-16)
    g2a = (a2a_ref[0] + a2a_ref[1]) / (za_ref[0] + za_ref[1] + 1e-16)
    hid = (_hi_dot(g2t, wg2t_ref[...]) + _hi_dot(g2a, wg2a_ref[...])
           + _hi_dot(ha_ref[...], wha_ref[...])
           + _hi_dot(ht_ref[...], wht_ref[...]) + b1_ref[...])
    hid = jnp.maximum(hid, 0.0)
    lg = _hi_dot(hid, wc2_ref[...]) + b2_ref[...]
    m = jnp.max(lg, axis=1, keepdims=True)
    p = lg - m
    out_ref[...] = p - jnp.log(jnp.sum(jnp.exp(p), axis=1, keepdims=True))

  return pl.pallas_call(
      body,
      grid=(NB,),
      in_specs=[
          pl.BlockSpec((NC, BR, h2), lambda i: (0, i, 0)),
          pl.BlockSpec((NC, BR, h2), lambda i: (0, i, 0)),
          pl.BlockSpec((NC, BR, 1), lambda i: (0, i, 0)),
          pl.BlockSpec((NC, BR, 1), lambda i: (0, i, 0)),
          pl.BlockSpec((BR, ut), lambda i: (i, 0)),
          pl.BlockSpec((BR, ua), lambda i: (i, 0)),
          pl.BlockSpec((h2, hc), lambda i: (0, 0)),
          pl.BlockSpec((h2, hc), lambda i: (0, 0)),
          pl.BlockSpec((ua, hc), lambda i: (0, 0)),
          pl.BlockSpec((ut, hc), lambda i: (0, 0)),
          pl.BlockSpec((1, hc), lambda i: (0, 0)),
          pl.BlockSpec((hc, tags), lambda i: (0, 0)),
          pl.BlockSpec((1, tags), lambda i: (0, 0)),
      ],
      out_specs=pl.BlockSpec((BR, tags), lambda i: (i, 0)),
      out_shape=jax.ShapeDtypeStruct((N, tags), F32),
  )(acc2_t, acc2_a, z3_t, z3_a, h_t, h_a,
    w_g2t, w_g2a, w_ha, w_ht, bc1r, wc2, bc2r)


def kernel(train_text, train_audio, edge_index, W_rnn_t, b_rnn_t, W_rnn_a,
           b_rnn_a, a_src_t, a_dst_t, a_src_a, a_dst_a, W1_t, W2_t, W1_a,
           W2_a, Wc1, bc1, Wc2, bc2):
  src3 = edge_index[0].reshape(NW, NCH, K)
  dst3 = edge_index[1].reshape(NW, NCH, K)
  xt = jnp.pad(train_text, ((0, NP - N), (0, 0)))
  xa = jnp.pad(train_audio, ((0, NP - N), (0, 0)))

  h_t, x1_t, sd_t = _encode(xt, W_rnn_t, b_rnn_t, a_src_t, a_dst_t, W1_t)
  h_a, x1_a, sd_a = _encode(xa, W_rnn_a, b_rnn_a, a_src_a, a_dst_a, W1_a)

  ex_t, z_t = _attn_kernel(sd_t[:, 0], sd_t[:, 1], src3, dst3)
  ex_a, z_a = _attn_kernel(sd_a[:, 0], sd_a[:, 1], src3, dst3)
  z3_t = z_t.reshape(NC, NP, 1)
  z3_a = z_a.reshape(NC, NP, 1)

  exbits_t = lax.bitcast_convert_type(ex_t.reshape(NW, NCH, K), jnp.int32)
  exbits_a = lax.bitcast_convert_type(ex_a.reshape(NW, NCH, K), jnp.int32)
  pack = jnp.stack([src3, dst3, exbits_t, exbits_a], axis=2)

  acc1_t = _msgpass_t(x1_t, pack)
  acc1_a = _msgpass_a(x1_a, pack)

  x2_t = _layer1(acc1_t, z3_t, W2_t)
  x2_a = _layer1(acc1_a, z3_a, W2_a)

  x2_cat = jnp.concatenate([x2_t, x2_a], axis=1)
  acc2 = _msgpass_cat(x2_cat, pack)
  acc2_t = acc2[:, :, 0:64]
  acc2_a = acc2[:, :, 64:128]

  return _final(acc2_t, acc2_a, z3_t, z3_a, h_t, h_a, Wc1, bc1, Wc2, bc2)


# speedup vs baseline: 1.3827x; 1.1499x over previous
import functools

import jax
import jax.numpy as jnp
from jax import lax
from jax.experimental import pallas as pl
from jax.experimental.pallas import tpu as pltpu
from jax.experimental.pallas import tpu_sc as plsc

N = 10000
NP = 10240
E = 320000
NC = 2
NS = 16
NW = NC * NS
EW = E // NW
K = 80
NCH = EW // K
RW = NP // NS
BR = 256
NB = NP // BR
F32 = jnp.float32

_mesh = plsc.VectorSubcoreMesh(core_axis_name="c", subcore_axis_name="s")


def _hi_dot(a, b):
  return lax.dot_general(a, b, (((1,), (0,)), ((), ())),
                         preferred_element_type=F32,
                         precision=lax.Precision.DEFAULT)


def _encode(x, wr, b, a_src, a_dst, w1):
  u = x.shape[1]
  h1 = w1.shape[1]
  a2 = jnp.stack([a_src, a_dst], axis=1)
  b2 = b.reshape(1, u)

  def body(x_ref, wr_ref, b_ref, a2_ref, w1_ref, h_ref, x1_ref, sd_ref):
    h = jnp.tanh(_hi_dot(x_ref[...], wr_ref[...]) + b_ref[...])
    h_ref[...] = h
    x1_ref[...] = _hi_dot(h, w1_ref[...])
    sd_ref[...] = _hi_dot(h, a2_ref[...])

  h, x1, sd = pl.pallas_call(
      body,
      grid=(NB,),
      in_specs=[
          pl.BlockSpec((BR, u), lambda i: (i, 0)),
          pl.BlockSpec((u, u), lambda i: (0, 0)),
          pl.BlockSpec((1, u), lambda i: (0, 0)),
          pl.BlockSpec((u, 2), lambda i: (0, 0)),
          pl.BlockSpec((u, h1), lambda i: (0, 0)),
      ],
      out_specs=[
          pl.BlockSpec((BR, u), lambda i: (i, 0)),
          pl.BlockSpec((BR, h1), lambda i: (i, 0)),
          pl.BlockSpec((BR, 2), lambda i: (i, 0)),
      ],
      out_shape=[
          jax.ShapeDtypeStruct((NP, u), F32),
          jax.ShapeDtypeStruct((NP, h1), F32),
          jax.ShapeDtypeStruct((NP, 2), F32),
      ],
  )(x, wr, b2, a2, w1)
  return h, x1, sd


@functools.partial(
    pl.kernel,
    out_type=[
        jax.ShapeDtypeStruct((NW, EW), F32),
        jax.ShapeDtypeStruct((NC, NP), F32),
    ],
    mesh=_mesh,
    scratch_types=[
        pltpu.VMEM((NP,), F32),
        pltpu.VMEM((NP,), F32),
        pltpu.VMEM((NCH, K), jnp.int32),
        pltpu.VMEM((NCH, K), jnp.int32),
        pltpu.VMEM((EW,), F32),
        pltpu.VMEM((RW,), F32),
        pltpu.VMEM((128,), F32),
        pltpu.VMEM_SHARED((NP,), F32),
    ],
    compiler_params=pltpu.CompilerParams(needs_layout_passes=False, use_tc_tiling_on_sc=False),
)
def _attn_kernel(s_hbm, d_hbm, src_hbm, dst_hbm, ex_out, z_out,
                 sv, dv, srcv, dstv, exbuf, zslice, tmp16, zsh):
  cid = lax.axis_index("c")
  sid = lax.axis_index("s")
  wid = sid * NC + cid
  pltpu.sync_copy(s_hbm, sv)
  pltpu.sync_copy(d_hbm, dv)
  pltpu.sync_copy(src_hbm.at[wid], srcv)
  pltpu.sync_copy(dst_hbm.at[wid], dstv)

  zero16 = jnp.zeros((16,), F32)
  for i in range(RW // 16):
    zslice[pl.ds(i * 16, 16)] = zero16
  pltpu.sync_copy(zslice, zsh.at[pl.ds(sid * RW, RW)])
  plsc.subcore_barrier()

  def maxbody(i, carry):
    ms, md = carry
    return (jnp.maximum(ms, sv[pl.ds(i * 16, 16)]),
            jnp.maximum(md, dv[pl.ds(i * 16, 16)]))

  ms, md = lax.fori_loop(0, NP // 16, maxbody,
                         (jnp.full((16,), -1e30, F32),
                          jnp.full((16,), -1e30, F32)))
  lanes = lax.iota(jnp.int32, 16)

  def lane_max(v):
    for shift in (8, 4, 2, 1):
      tmp16[pl.ds(0, 16)] = v
      v = jnp.maximum(v, plsc.load_gather(tmp16, [(lanes + shift) & 15]))
    return v

  csplat = jnp.maximum(lane_max(ms) + lane_max(md), 0.0)

  def chunk(c, carry):
    for j in range(K // 16):
      si = srcv[c, pl.ds(j * 16, 16)]
      di = dstv[c, pl.ds(j * 16, 16)]
      e = plsc.load_gather(sv, [si]) + plsc.load_gather(dv, [di])
      e = jnp.where(e >= 0.0, e, 0.2 * e)
      exbuf[pl.ds(c * K + j * 16, 16)] = jnp.exp(e - csplat)
    pltpu.sync_copy(exbuf.at[pl.ds(c * K, K)], zsh.at[dstv.at[c]], add=True)
    return carry

  lax.fori_loop(0, NCH, chunk, 0)
  pltpu.sync_copy(exbuf, ex_out.at[wid])
  plsc.subcore_barrier()
  pltpu.sync_copy(zsh.at[pl.ds(sid * RW, RW)],
                  z_out.at[cid, pl.ds(sid * RW, RW)])


def _make_msgpass(d, segs):
  @functools.partial(
      pl.kernel,
      out_type=jax.ShapeDtypeStruct((NC, NP, d), F32),
      mesh=_mesh,
      scratch_types=[
          [pltpu.VMEM((4, K), jnp.int32)] * 4,
          [pltpu.VMEM((K, d), F32)] * 2,
          pltpu.VMEM((16, d), F32),
          pltpu.VMEM_SHARED((NP, d), F32),
          [pltpu.SemaphoreType.DMA] * 4,
          [pltpu.SemaphoreType.DMA] * 2,
          [pltpu.SemaphoreType.DMA] * 2,
      ],
      compiler_params=pltpu.CompilerParams(needs_layout_passes=False, use_tc_tiling_on_sc=False),
  )
  def msg_kernel(x_hbm, pack_hbm, acc_out,
                 idxb, rowsb, zrow, acc, isem, gsem, ssem):
    cid = lax.axis_index("c")
    sid = lax.axis_index("s")
    wid = sid * NC + cid

    zero16 = jnp.zeros((16,), F32)
    for i in range(16):
      for j in range(d // 16):
        zrow[i, pl.ds(j * 16, 16)] = zero16
    for i in range(RW // 16):
      pltpu.sync_copy(zrow, acc.at[pl.ds(sid * RW + i * 16, 16)])
    plsc.subcore_barrier()

    def start_idx(c, ib):
      pltpu.async_copy(pack_hbm.at[wid, c], idxb[ib], isem[ib])

    def wait_idx(ib):
      pltpu.make_async_copy(pack_hbm.at[wid, 0], idxb[ib], isem[ib]).wait()

    def start_gather(ib, b):
      pltpu.async_copy(x_hbm.at[idxb[ib].at[0]], rowsb[b], gsem[b])

    def wait_gather(ib, b):
      pltpu.make_async_copy(x_hbm.at[idxb[ib].at[0]], rowsb[b], gsem[b]).wait()

    def start_scatter(ib, b):
      pltpu.async_copy(rowsb[b], acc.at[idxb[ib].at[1]], ssem[b], add=True)

    def wait_scatter(ib, b):
      pltpu.make_async_copy(rowsb[b], acc.at[idxb[ib].at[1]], ssem[b]).wait()

    def scale(ib, b):
      rows = rowsb[b]

      def sbody(k0, carry):
        base = k0 * 16
        for kk in range(16):
          for (prow, jlo, jhi) in segs:
            wbits = plsc.load_gather(
                idxb[ib], [jnp.full((16,), prow, jnp.int32),
                           jnp.full((16,), base + kk, jnp.int32)])
            w = plsc.bitcast(wbits, F32)
            for j in range(jlo, jhi):
              rows[base + kk, pl.ds(j * 16, 16)] = (
                  rows[base + kk, pl.ds(j * 16, 16)] * w)
        return carry

      lax.fori_loop(0, K // 16, sbody, 0)

    def step(c_dyn, cm, first, has_next, has_next2):
      ib, b = cm % 4, cm % 2
      wait_gather(ib, b)
      scale(ib, b)
      start_scatter(ib, b)
      if not first:
        wait_scatter((ib - 1) % 4, b ^ 1)
      if has_next:
        wait_idx((ib + 1) % 4)
        start_gather((ib + 1) % 4, b ^ 1)
      if has_next2:
        start_idx(c_dyn + 2, (ib + 2) % 4)

    start_idx(0, 0)
    start_idx(1, 1)
    wait_idx(0)
    start_gather(0, 0)
    step(0, 0, True, True, True)
    step(1, 1, False, True, True)
    step(2, 2, False, True, True)

    def body(t, carry):
      c0 = 4 * t + 3
      step(c0, 3, False, True, True)
      step(c0 + 1, 0, False, True, True)
      step(c0 + 2, 1, False, True, True)
      step(c0 + 3, 2, False, True, True)
      return carry

    lax.fori_loop(0, (NCH - 5) // 4, body, 0)
    step(NCH - 2, (NCH - 2) % 4, False, True, False)
    step(NCH - 1, (NCH - 1) % 4, False, False, False)
    wait_scatter((NCH - 1) % 4, (NCH - 1) % 2)

    plsc.subcore_barrier()
    pltpu.sync_copy(acc.at[pl.ds(sid * RW, RW)],
                    acc_out.at[cid, pl.ds(sid * RW, RW)])

  return msg_kernel


_msgpass_t = _make_msgpass(128, ((2, 0, 8),))
_msgpass_a = _make_msgpass(128, ((3, 0, 8),))
_msgpass_cat = _make_msgpass(128, ((2, 0, 4), (3, 4, 8)))


def _layer1(acc, z3, w2):
  h1, h2 = w2.shape

  def body(acc_ref, z_ref, w2_ref, x2_ref):
    den = z_ref[0] + z_ref[1] + 1e-16
    g = jnp.maximum((acc_ref[0] + acc_ref[1]) / den, 0.0)
    x2_ref[...] = _hi_dot(g, w2_ref[...])

  return pl.pallas_call(
      body,
      grid=(NB,),
      in_specs=[
          pl.BlockSpec((NC, BR, h1), lambda i: (0, i, 0)),
          pl.BlockSpec((NC, BR, 1), lambda i: (0, i, 0)),
          pl.BlockSpec((h1, h2), lambda i: (0, 0)),
      ],
      out_specs=pl.BlockSpec((BR, h2), lambda i: (i, 0)),
      out_shape=jax.ShapeDtypeStruct((NP, h2), F32),
  )(acc, z3, w2)


def _final(acc2_t, acc2_a, z3_t, z3_a, h_t, h_a, wc1, bc1, wc2, bc2):
  h2 = acc2_t.shape[2]
  ut = h_t.shape[1]
  ua = h_a.shape[1]
  hc = wc1.shape[1]
  tags = wc2.shape[1]
  w_g2t = wc1[0:h2]
  w_g2a = wc1[h2:2 * h2]
  w_ha = wc1[2 * h2:2 * h2 + ua]
  w_ht = wc1[2 * h2 + ua:]
  bc1r = bc1.reshape(1, hc)
  bc2r = bc2.reshape(1, tags)

  def body(a2t_ref, a2a_ref, zt_ref, za_ref, ht_ref, ha_ref,
           wg2t_ref, wg2a_ref, wha_ref, wht_ref, b1_ref, wc2_ref, b2_ref,
           out_ref):
    g2t = (a2t_ref[0] + a2t_ref[1]) / (zt_ref[0] + zt_ref[1] + 1e-16)
    g2a = (a2a_ref[0] + a2a_ref[1]) / (za_ref[0] + za_ref[1] + 1e-16)
    hid = (_hi_dot(g2t, wg2t_ref[...]) + _hi_dot(g2a, wg2a_ref[...])
           + _hi_dot(ha_ref[...], wha_ref[...])
           + _hi_dot(ht_ref[...], wht_ref[...]) + b1_ref[...])
    hid = jnp.maximum(hid, 0.0)
    lg = _hi_dot(hid, wc2_ref[...]) + b2_ref[...]
    m = jnp.max(lg, axis=1, keepdims=True)
    p = lg - m
    out_ref[...] = p - jnp.log(jnp.sum(jnp.exp(p), axis=1, keepdims=True))

  return pl.pallas_call(
      body,
      grid=(NB,),
      in_specs=[
          pl.BlockSpec((NC, BR, h2), lambda i: (0, i, 0)),
          pl.BlockSpec((NC, BR, h2), lambda i: (0, i, 0)),
          pl.BlockSpec((NC, BR, 1), lambda i: (0, i, 0)),
          pl.BlockSpec((NC, BR, 1), lambda i: (0, i, 0)),
          pl.BlockSpec((BR, ut), lambda i: (i, 0)),
          pl.BlockSpec((BR, ua), lambda i: (i, 0)),
          pl.BlockSpec((h2, hc), lambda i: (0, 0)),
          pl.BlockSpec((h2, hc), lambda i: (0, 0)),
          pl.BlockSpec((ua, hc), lambda i: (0, 0)),
          pl.BlockSpec((ut, hc), lambda i: (0, 0)),
          pl.BlockSpec((1, hc), lambda i: (0, 0)),
          pl.BlockSpec((hc, tags), lambda i: (0, 0)),
          pl.BlockSpec((1, tags), lambda i: (0, 0)),
      ],
      out_specs=pl.BlockSpec((BR, tags), lambda i: (i, 0)),
      out_shape=jax.ShapeDtypeStruct((N, tags), F32),
  )(acc2_t, acc2_a, z3_t, z3_a, h_t, h_a,
    w_g2t, w_g2a, w_ha, w_ht, bc1r, wc2, bc2r)


def kernel(train_text, train_audio, edge_index, W_rnn_t, b_rnn_t, W_rnn_a,
           b_rnn_a, a_src_t, a_dst_t, a_src_a, a_dst_a, W1_t, W2_t, W1_a,
           W2_a, Wc1, bc1, Wc2, bc2):
  src3 = edge_index[0].reshape(NW, NCH, K)
  dst3 = edge_index[1].reshape(NW, NCH, K)
  xt = jnp.pad(train_text, ((0, NP - N), (0, 0)))
  xa = jnp.pad(train_audio, ((0, NP - N), (0, 0)))

  h_t, x1_t, sd_t = _encode(xt, W_rnn_t, b_rnn_t, a_src_t, a_dst_t, W1_t)
  h_a, x1_a, sd_a = _encode(xa, W_rnn_a, b_rnn_a, a_src_a, a_dst_a, W1_a)

  ex_t, z_t = _attn_kernel(sd_t[:, 0], sd_t[:, 1], src3, dst3)
  ex_a, z_a = _attn_kernel(sd_a[:, 0], sd_a[:, 1], src3, dst3)
  z3_t = z_t.reshape(NC, NP, 1)
  z3_a = z_a.reshape(NC, NP, 1)

  exbits_t = lax.bitcast_convert_type(ex_t.reshape(NW, NCH, K), jnp.int32)
  exbits_a = lax.bitcast_convert_type(ex_a.reshape(NW, NCH, K), jnp.int32)
  pack = jnp.stack([src3, dst3, exbits_t, exbits_a], axis=2)

  acc1_t = _msgpass_t(x1_t, pack)
  acc1_a = _msgpass_a(x1_a, pack)

  x2_t = _layer1(acc1_t, z3_t, W2_t)
  x2_a = _layer1(acc1_a, z3_a, W2_a)

  x2_cat = jnp.concatenate([x2_t, x2_a], axis=1)
  acc2 = _msgpass_cat(x2_cat, pack)
  acc2_t = acc2[:, :, 0:64]
  acc2_a = acc2[:, :, 64:128]

  return _final(acc2_t, acc2_a, z3_t, z3_a, h_t, h_a, Wc1, bc1, Wc2, bc2)


# speedup vs baseline: 1.3859x; 1.0023x over previous
import functools

import jax
import jax.numpy as jnp
from jax import lax
from jax.experimental import pallas as pl
from jax.experimental.pallas import tpu as pltpu
from jax.experimental.pallas import tpu_sc as plsc

N = 10000
NP = 10240
E = 320000
NC = 2
NS = 16
NW = NC * NS
EW = E // NW
K = 80
NCH = EW // K
RW = NP // NS
BR = 256
NB = NP // BR
F32 = jnp.float32

_mesh = plsc.VectorSubcoreMesh(core_axis_name="c", subcore_axis_name="s")


def _hi_dot(a, b):
  return lax.dot_general(a, b, (((1,), (0,)), ((), ())),
                         preferred_element_type=F32,
                         precision=lax.Precision.DEFAULT)


def _encode(x, wr, b, a_src, a_dst, w1):
  u = x.shape[1]
  h1 = w1.shape[1]
  a2 = jnp.stack([a_src, a_dst], axis=1)
  b2 = b.reshape(1, u)

  def body(x_ref, wr_ref, b_ref, a2_ref, w1_ref, h_ref, x1_ref, sd_ref):
    h = jnp.tanh(_hi_dot(x_ref[...], wr_ref[...]) + b_ref[...])
    h_ref[...] = h
    x1_ref[...] = _hi_dot(h, w1_ref[...])
    sd_ref[...] = _hi_dot(h, a2_ref[...])

  h, x1, sd = pl.pallas_call(
      body,
      grid=(NB,),
      in_specs=[
          pl.BlockSpec((BR, u), lambda i: (i, 0)),
          pl.BlockSpec((u, u), lambda i: (0, 0)),
          pl.BlockSpec((1, u), lambda i: (0, 0)),
          pl.BlockSpec((u, 2), lambda i: (0, 0)),
          pl.BlockSpec((u, h1), lambda i: (0, 0)),
      ],
      out_specs=[
          pl.BlockSpec((BR, u), lambda i: (i, 0)),
          pl.BlockSpec((BR, h1), lambda i: (i, 0)),
          pl.BlockSpec((BR, 2), lambda i: (i, 0)),
      ],
      out_shape=[
          jax.ShapeDtypeStruct((NP, u), F32),
          jax.ShapeDtypeStruct((NP, h1), F32),
          jax.ShapeDtypeStruct((NP, 2), F32),
      ],
  )(x, wr, b2, a2, w1)
  return h, x1, sd


@functools.partial(
    pl.kernel,
    out_type=[
        jax.ShapeDtypeStruct((NW, NCH, 4, K), F32),
        jax.ShapeDtypeStruct((NC, NP), F32),
        jax.ShapeDtypeStruct((NC, NP), F32),
    ],
    mesh=_mesh,
    scratch_types=[
        pltpu.VMEM((NP,), F32),
        pltpu.VMEM((NP,), F32),
        pltpu.VMEM((NP,), F32),
        pltpu.VMEM((NP,), F32),
        pltpu.VMEM((NCH, K), jnp.int32),
        pltpu.VMEM((NCH, K), jnp.int32),
        pltpu.VMEM((NCH, 4, K), F32),
        pltpu.VMEM((RW,), F32),
        pltpu.VMEM((128,), F32),
        pltpu.VMEM_SHARED((NP,), F32),
        pltpu.VMEM_SHARED((NP,), F32),
        [pltpu.SemaphoreType.DMA] * 2,
        [pltpu.SemaphoreType.DMA] * 2,
    ],
    compiler_params=pltpu.CompilerParams(needs_layout_passes=False, use_tc_tiling_on_sc=False),
)
def _attn_kernel(st_hbm, dt_hbm, sa_hbm, da_hbm, src_hbm, dst_hbm,
                 pack_out, zt_out, za_out,
                 sv_t, dv_t, sv_a, dv_a, srcv, dstv, packv, zslice, tmp16,
                 zsh_t, zsh_a, ztsem, zasem):
  cid = lax.axis_index("c")
  sid = lax.axis_index("s")
  wid = sid * NC + cid
  pltpu.sync_copy(st_hbm, sv_t)
  pltpu.sync_copy(dt_hbm, dv_t)
  pltpu.sync_copy(sa_hbm, sv_a)
  pltpu.sync_copy(da_hbm, dv_a)
  pltpu.sync_copy(src_hbm.at[wid], srcv)
  pltpu.sync_copy(dst_hbm.at[wid], dstv)

  zero16 = jnp.zeros((16,), F32)
  for i in range(RW // 16):
    zslice[pl.ds(i * 16, 16)] = zero16
  pltpu.sync_copy(zslice, zsh_t.at[pl.ds(sid * RW, RW)])
  pltpu.sync_copy(zslice, zsh_a.at[pl.ds(sid * RW, RW)])
  plsc.subcore_barrier()

  def maxbody(i, carry):
    mst, mdt, msa, mda = carry
    return (jnp.maximum(mst, sv_t[pl.ds(i * 16, 16)]),
            jnp.maximum(mdt, dv_t[pl.ds(i * 16, 16)]),
            jnp.maximum(msa, sv_a[pl.ds(i * 16, 16)]),
            jnp.maximum(mda, dv_a[pl.ds(i * 16, 16)]))

  neg = jnp.full((16,), -1e30, F32)
  mst, mdt, msa, mda = lax.fori_loop(0, NP // 16, maxbody,
                                     (neg, neg, neg, neg))
  lanes = lax.iota(jnp.int32, 16)

  def lane_max(v):
    for shift in (8, 4, 2, 1):
      tmp16[pl.ds(0, 16)] = v
      v = jnp.maximum(v, plsc.load_gather(tmp16, [(lanes + shift) & 15]))
    return v

  csplat_t = jnp.maximum(lane_max(mst) + lane_max(mdt), 0.0)
  csplat_a = jnp.maximum(lane_max(msa) + lane_max(mda), 0.0)

  def leaky_exp(sv, dv, si, di, csplat):
    e = plsc.load_gather(sv, [si]) + plsc.load_gather(dv, [di])
    e = jnp.where(e >= 0.0, e, 0.2 * e)
    return jnp.exp(e - csplat)

  def compute(c):
    for j in range(K // 16):
      si = srcv[c, pl.ds(j * 16, 16)]
      di = dstv[c, pl.ds(j * 16, 16)]
      packv[c, 0, pl.ds(j * 16, 16)] = plsc.bitcast(si, F32)
      packv[c, 1, pl.ds(j * 16, 16)] = plsc.bitcast(di, F32)
      packv[c, 2, pl.ds(j * 16, 16)] = leaky_exp(sv_t, dv_t, si, di, csplat_t)
      packv[c, 3, pl.ds(j * 16, 16)] = leaky_exp(sv_a, dv_a, si, di, csplat_a)

  def fire_z(c, p):
    pltpu.async_copy(packv.at[c, 2], zsh_t.at[dstv.at[c]], ztsem[p],
                     add=True)
    pltpu.async_copy(packv.at[c, 3], zsh_a.at[dstv.at[c]], zasem[p],
                     add=True)

  def wait_z(c, p):
    pltpu.make_async_copy(packv.at[c, 2], zsh_t.at[dstv.at[c]],
                          ztsem[p]).wait()
    pltpu.make_async_copy(packv.at[c, 3], zsh_a.at[dstv.at[c]],
                          zasem[p]).wait()

  compute(0)
  fire_z(0, 0)
  compute(1)
  fire_z(1, 1)

  def chunk2(t, carry):
    c0 = 2 * t + 2
    compute(c0)
    wait_z(c0 - 2, 0)
    fire_z(c0, 0)
    compute(c0 + 1)
    wait_z(c0 - 1, 1)
    fire_z(c0 + 1, 1)
    return carry

  lax.fori_loop(0, (NCH - 3) // 2, chunk2, 0)
  compute(NCH - 1)
  wait_z(NCH - 3, 0)
  fire_z(NCH - 1, 0)
  wait_z(NCH - 2, 1)
  wait_z(NCH - 1, 0)

  pltpu.sync_copy(packv, pack_out.at[wid])
  plsc.subcore_barrier()
  pltpu.sync_copy(zsh_t.at[pl.ds(sid * RW, RW)],
                  zt_out.at[cid, pl.ds(sid * RW, RW)])
  pltpu.sync_copy(zsh_a.at[pl.ds(sid * RW, RW)],
                  za_out.at[cid, pl.ds(sid * RW, RW)])


def _make_msgpass(d, segs):
  @functools.partial(
      pl.kernel,
      out_type=jax.ShapeDtypeStruct((NC, NP, d), F32),
      mesh=_mesh,
      scratch_types=[
          [pltpu.VMEM((4, K), jnp.int32)] * 4,
          [pltpu.VMEM((K, d), F32)] * 2,
          pltpu.VMEM((16, d), F32),
          pltpu.VMEM_SHARED((NP, d), F32),
          [pltpu.SemaphoreType.DMA] * 4,
          [pltpu.SemaphoreType.DMA] * 2,
          [pltpu.SemaphoreType.DMA] * 2,
      ],
      compiler_params=pltpu.CompilerParams(needs_layout_passes=False, use_tc_tiling_on_sc=False),
  )
  def msg_kernel(x_hbm, pack_hbm, acc_out,
                 idxb, rowsb, zrow, acc, isem, gsem, ssem):
    cid = lax.axis_index("c")
    sid = lax.axis_index("s")
    wid = sid * NC + cid

    zero16 = jnp.zeros((16,), F32)
    for i in range(16):
      for j in range(d // 16):
        zrow[i, pl.ds(j * 16, 16)] = zero16
    for i in range(RW // 16):
      pltpu.sync_copy(zrow, acc.at[pl.ds(sid * RW + i * 16, 16)])
    plsc.subcore_barrier()

    def start_idx(c, ib):
      pltpu.async_copy(pack_hbm.at[wid, c], idxb[ib], isem[ib])

    def wait_idx(ib):
      pltpu.make_async_copy(pack_hbm.at[wid, 0], idxb[ib], isem[ib]).wait()

    def start_gather(ib, b):
      pltpu.async_copy(x_hbm.at[idxb[ib].at[0]], rowsb[b], gsem[b])

    def wait_gather(ib, b):
      pltpu.make_async_copy(x_hbm.at[idxb[ib].at[0]], rowsb[b], gsem[b]).wait()

    def start_scatter(ib, b):
      pltpu.async_copy(rowsb[b], acc.at[idxb[ib].at[1]], ssem[b], add=True)

    def wait_scatter(ib, b):
      pltpu.make_async_copy(rowsb[b], acc.at[idxb[ib].at[1]], ssem[b]).wait()

    def scale(ib, b):
      rows = rowsb[b]

      def sbody(k0, carry):
        base = k0 * 16
        for kk in range(16):
          for (prow, jlo, jhi) in segs:
            wbits = plsc.load_gather(
                idxb[ib], [jnp.full((16,), prow, jnp.int32),
                           jnp.full((16,), base + kk, jnp.int32)])
            w = plsc.bitcast(wbits, F32)
            for j in range(jlo, jhi):
              rows[base + kk, pl.ds(j * 16, 16)] = (
                  rows[base + kk, pl.ds(j * 16, 16)] * w)
        return carry

      lax.fori_loop(0, K // 16, sbody, 0)

    def step(c_dyn, cm, first, has_next, has_next2):
      ib, b = cm % 4, cm % 2
      wait_gather(ib, b)
      scale(ib, b)
      start_scatter(ib, b)
      if not first:
        wait_scatter((ib - 1) % 4, b ^ 1)
      if has_next:
        wait_idx((ib + 1) % 4)
        start_gather((ib + 1) % 4, b ^ 1)
      if has_next2:
        start_idx(c_dyn + 2, (ib + 2) % 4)

    start_idx(0, 0)
    start_idx(1, 1)
    wait_idx(0)
    start_gather(0, 0)
    step(0, 0, True, True, True)
    step(1, 1, False, True, True)
    step(2, 2, False, True, True)

    def body(t, carry):
      c0 = 4 * t + 3
      step(c0, 3, False, True, True)
      step(c0 + 1, 0, False, True, True)
      step(c0 + 2, 1, False, True, True)
      step(c0 + 3, 2, False, True, True)
      return carry

    lax.fori_loop(0, (NCH - 5) // 4, body, 0)
    step(NCH - 2, (NCH - 2) % 4, False, True, False)
    step(NCH - 1, (NCH - 1) % 4, False, False, False)
    wait_scatter((NCH - 1) % 4, (NCH - 1) % 2)

    plsc.subcore_barrier()
    pltpu.sync_copy(acc.at[pl.ds(sid * RW, RW)],
                    acc_out.at[cid, pl.ds(sid * RW, RW)])

  return msg_kernel


_msgpass_t = _make_msgpass(128, ((2, 0, 8),))
_msgpass_a = _make_msgpass(128, ((3, 0, 8),))
_msgpass_cat = _make_msgpass(128, ((2, 0, 4), (3, 4, 8)))


def _layer1(acc, z3, w2):
  h1, h2 = w2.shape

  def body(acc_ref, z_ref, w2_ref, x2_ref):
    den = z_ref[0] + z_ref[1] + 1e-16
    g = jnp.maximum((acc_ref[0] + acc_ref[1]) / den, 0.0)
    x2_ref[...] = _hi_dot(g, w2_ref[...])

  return pl.pallas_call(
      body,
      grid=(NB,),
      in_specs=[
          pl.BlockSpec((NC, BR, h1), lambda i: (0, i, 0)),
          pl.BlockSpec((NC, BR, 1), lambda i: (0, i, 0)),
          pl.BlockSpec((h1, h2), lambda i: (0, 0)),
      ],
      out_specs=pl.BlockSpec((BR, h2), lambda i: (i, 0)),
      out_shape=jax.ShapeDtypeStruct((NP, h2), F32),
  )(acc, z3, w2)


def _final(acc2_t, acc2_a, z3_t, z3_a, h_t, h_a, wc1, bc1, wc2, bc2):
  h2 = acc2_t.shape[2]
  ut = h_t.shape[1]
  ua = h_a.shape[1]
  hc = wc1.shape[1]
  tags = wc2.shape[1]
  w_g2t = wc1[0:h2]
  w_g2a = wc1[h2:2 * h2]
  w_ha = wc1[2 * h2:2 * h2 + ua]
  w_ht = wc1[2 * h2 + ua:]
  bc1r = bc1.reshape(1, hc)
  bc2r = bc2.reshape(1, tags)

  def body(a2t_ref, a2a_ref, zt_ref, za_ref, ht_ref, ha_ref,
           wg2t_ref, wg2a_ref, wha_ref, wht_ref, b1_ref, wc2_ref, b2_ref,
           out_ref):
    g2t = (a2t_ref[0] + a2t_ref[1]) / (zt_ref[0] + zt_ref[1] + 1e-16)
    g2a = (a2a_ref[0] + a2a_ref[1]) / (za_ref[0] + za_ref[1] + 1e-16)
    hid = (_hi_dot(g2t, wg2t_ref[...]) + _hi_dot(g2a, wg2a_ref[...])
           + _hi_dot(ha_ref[...], wha_ref[...])
           + _hi_dot(ht_ref[...], wht_ref[...]) + b1_ref[...])
    hid = jnp.maximum(hid, 0.0)
    lg = _hi_dot(hid, wc2_ref[...]) + b2_ref[...]
    m = jnp.max(lg, axis=1, keepdims=True)
    p = lg - m
    out_ref[...] = p - jnp.log(jnp.sum(jnp.exp(p), axis=1, keepdims=True))

  return pl.pallas_call(
      body,
      grid=(NB,),
      in_specs=[
          pl.BlockSpec((NC, BR, h2), lambda i: (0, i, 0)),
          pl.BlockSpec((NC, BR, h2), lambda i: (0, i, 0)),
          pl.BlockSpec((NC, BR, 1), lambda i: (0, i, 0)),
          pl.BlockSpec((NC, BR, 1), lambda i: (0, i, 0)),
          pl.BlockSpec((BR, ut), lambda i: (i, 0)),
          pl.BlockSpec((BR, ua), lambda i: (i, 0)),
          pl.BlockSpec((h2, hc), lambda i: (0, 0)),
          pl.BlockSpec((h2, hc), lambda i: (0, 0)),
          pl.BlockSpec((ua, hc), lambda i: (0, 0)),
          pl.BlockSpec((ut, hc), lambda i: (0, 0)),
          pl.BlockSpec((1, hc), lambda i: (0, 0)),
          pl.BlockSpec((hc, tags), lambda i: (0, 0)),
          pl.BlockSpec((1, tags), lambda i: (0, 0)),
      ],
      out_specs=pl.BlockSpec((BR, tags), lambda i: (i, 0)),
      out_shape=jax.ShapeDtypeStruct((N, tags), F32),
  )(acc2_t, acc2_a, z3_t, z3_a, h_t, h_a,
    w_g2t, w_g2a, w_ha, w_ht, bc1r, wc2, bc2r)


def kernel(train_text, train_audio, edge_index, W_rnn_t, b_rnn_t, W_rnn_a,
           b_rnn_a, a_src_t, a_dst_t, a_src_a, a_dst_a, W1_t, W2_t, W1_a,
           W2_a, Wc1, bc1, Wc2, bc2):
  src3 = edge_index[0].reshape(NW, NCH, K)
  dst3 = edge_index[1].reshape(NW, NCH, K)
  xt = jnp.pad(train_text, ((0, NP - N), (0, 0)))
  xa = jnp.pad(train_audio, ((0, NP - N), (0, 0)))

  h_t, x1_t, sd_t = _encode(xt, W_rnn_t, b_rnn_t, a_src_t, a_dst_t, W1_t)
  h_a, x1_a, sd_a = _encode(xa, W_rnn_a, b_rnn_a, a_src_a, a_dst_a, W1_a)

  pack_f, z_t, z_a = _attn_kernel(sd_t[:, 0], sd_t[:, 1],
                                  sd_a[:, 0], sd_a[:, 1], src3, dst3)
  pack = lax.bitcast_convert_type(pack_f, jnp.int32)
  z3_t = z_t.reshape(NC, NP, 1)
  z3_a = z_a.reshape(NC, NP, 1)

  acc1_t = _msgpass_t(x1_t, pack)
  acc1_a = _msgpass_a(x1_a, pack)

  x2_t = _layer1(acc1_t, z3_t, W2_t)
  x2_a = _layer1(acc1_a, z3_a, W2_a)

  x2_cat = jnp.concatenate([x2_t, x2_a], axis=1)
  acc2 = _msgpass_cat(x2_cat, pack)
  acc2_t = acc2[:, :, 0:64]
  acc2_a = acc2[:, :, 64:128]

  return _final(acc2_t, acc2_a, z3_t, z3_a, h_t, h_a, Wc1, bc1, Wc2, bc2)


# speedup vs baseline: 1.4994x; 1.0819x over previous
import functools

import jax
import jax.numpy as jnp
from jax import lax
from jax.experimental import pallas as pl
from jax.experimental.pallas import tpu as pltpu
from jax.experimental.pallas import tpu_sc as plsc

N = 10000
NP = 10240
E = 320000
NC = 2
NS = 16
NW = NC * NS
EW = E // NW
K = 80
NCH = EW // K
RW = NP // NS
BR = 256
NB = NP // BR
F32 = jnp.float32

_mesh = plsc.VectorSubcoreMesh(core_axis_name="c", subcore_axis_name="s")


def _hi_dot(a, b):
  return lax.dot_general(a, b, (((1,), (0,)), ((), ())),
                         preferred_element_type=F32,
                         precision=lax.Precision.DEFAULT)


def _encode(x, wr, b, a_src, a_dst, w1):
  u = x.shape[1]
  h1 = w1.shape[1]
  a2 = jnp.stack([a_src, a_dst], axis=1)
  b2 = b.reshape(1, u)

  def body(x_ref, wr_ref, b_ref, a2_ref, w1_ref, h_ref, x1_ref, sd_ref):
    h = jnp.tanh(_hi_dot(x_ref[...], wr_ref[...]) + b_ref[...])
    h_ref[...] = h
    x1_ref[...] = _hi_dot(h, w1_ref[...])
    sd_ref[...] = lax.dot_general(a2_ref[...], h, (((0,), (1,)), ((), ())),
                                  preferred_element_type=F32,
                                  precision=lax.Precision.DEFAULT)

  h, x1, sd = pl.pallas_call(
      body,
      grid=(NB,),
      in_specs=[
          pl.BlockSpec((BR, u), lambda i: (i, 0)),
          pl.BlockSpec((u, u), lambda i: (0, 0)),
          pl.BlockSpec((1, u), lambda i: (0, 0)),
          pl.BlockSpec((u, 2), lambda i: (0, 0)),
          pl.BlockSpec((u, h1), lambda i: (0, 0)),
      ],
      out_specs=[
          pl.BlockSpec((BR, u), lambda i: (i, 0)),
          pl.BlockSpec((BR, h1), lambda i: (i, 0)),
          pl.BlockSpec((2, BR), lambda i: (0, i)),
      ],
      out_shape=[
          jax.ShapeDtypeStruct((N, u), F32),
          jax.ShapeDtypeStruct((N, h1), F32),
          jax.ShapeDtypeStruct((2, N), F32),
      ],
  )(x, wr, b2, a2, w1)
  return h, x1, sd


@functools.partial(
    pl.kernel,
    out_type=[
        jax.ShapeDtypeStruct((NW, NCH, 4, K), jnp.int32),
        jax.ShapeDtypeStruct((NC, NP), F32),
        jax.ShapeDtypeStruct((NC, NP), F32),
    ],
    mesh=_mesh,
    scratch_types=[
        pltpu.VMEM((2, N), F32),
        pltpu.VMEM((2, N), F32),
        pltpu.VMEM((NCH, K), jnp.int32),
        pltpu.VMEM((NCH, K), jnp.int32),
        pltpu.VMEM((NCH, 4, K), jnp.int32),
        pltpu.VMEM((2, K), F32),
        pltpu.VMEM((2, K), F32),
        pltpu.VMEM((RW,), F32),
        pltpu.VMEM((128,), F32),
        pltpu.VMEM_SHARED((NP,), F32),
        pltpu.VMEM_SHARED((NP,), F32),
        [pltpu.SemaphoreType.DMA] * 2,
        [pltpu.SemaphoreType.DMA] * 2,
    ],
    compiler_params=pltpu.CompilerParams(needs_layout_passes=False, use_tc_tiling_on_sc=False),
)
def _attn_kernel(sdt_hbm, sda_hbm, src_hbm, dst_hbm,
                 pack_out, zt_out, za_out,
                 sdv_t, sdv_a, srcv, dstv, packv, exz_t, exz_a,
                 zslice, tmp16, zsh_t, zsh_a, ztsem, zasem):
  cid = lax.axis_index("c")
  sid = lax.axis_index("s")
  wid = sid * NC + cid
  pltpu.sync_copy(sdt_hbm, sdv_t)
  pltpu.sync_copy(sda_hbm, sdv_a)
  pltpu.sync_copy(src_hbm.at[wid], srcv)
  pltpu.sync_copy(dst_hbm.at[wid], dstv)

  zero16 = jnp.zeros((16,), F32)
  for i in range(RW // 16):
    zslice[pl.ds(i * 16, 16)] = zero16
  pltpu.sync_copy(zslice, zsh_t.at[pl.ds(sid * RW, RW)])
  pltpu.sync_copy(zslice, zsh_a.at[pl.ds(sid * RW, RW)])
  plsc.subcore_barrier()

  def maxbody(i, carry):
    mst, mdt, msa, mda = carry
    return (jnp.maximum(mst, sdv_t[0, pl.ds(i * 16, 16)]),
            jnp.maximum(mdt, sdv_t[1, pl.ds(i * 16, 16)]),
            jnp.maximum(msa, sdv_a[0, pl.ds(i * 16, 16)]),
            jnp.maximum(mda, sdv_a[1, pl.ds(i * 16, 16)]))

  neg = jnp.full((16,), -1e30, F32)
  mst, mdt, msa, mda = lax.fori_loop(0, N // 16, maxbody,
                                     (neg, neg, neg, neg))
  lanes = lax.iota(jnp.int32, 16)

  def lane_max(v):
    for shift in (8, 4, 2, 1):
      tmp16[pl.ds(0, 16)] = v
      v = jnp.maximum(v, plsc.load_gather(tmp16, [(lanes + shift) & 15]))
    return v

  csplat_t = jnp.maximum(lane_max(mst) + lane_max(mdt), 0.0)
  csplat_a = jnp.maximum(lane_max(msa) + lane_max(mda), 0.0)

  zero16i = jnp.zeros((16,), jnp.int32)
  one16i = jnp.ones((16,), jnp.int32)

  def leaky_exp(sdv, si, di, csplat):
    e = (plsc.load_gather(sdv, [zero16i, si])
         + plsc.load_gather(sdv, [one16i, di]))
    e = jnp.where(e >= 0.0, e, 0.2 * e)
    return jnp.exp(e - csplat)

  def compute(c, p):
    for j in range(K // 16):
      si = srcv[c, pl.ds(j * 16, 16)]
      di = dstv[c, pl.ds(j * 16, 16)]
      packv[c, 0, pl.ds(j * 16, 16)] = si
      packv[c, 1, pl.ds(j * 16, 16)] = di
      ext = leaky_exp(sdv_t, si, di, csplat_t)
      exa = leaky_exp(sdv_a, si, di, csplat_a)
      packv[c, 2, pl.ds(j * 16, 16)] = plsc.bitcast(ext, jnp.int32)
      packv[c, 3, pl.ds(j * 16, 16)] = plsc.bitcast(exa, jnp.int32)
      exz_t[p, pl.ds(j * 16, 16)] = ext
      exz_a[p, pl.ds(j * 16, 16)] = exa

  def fire_z(c, p):
    pltpu.async_copy(exz_t.at[p], zsh_t.at[dstv.at[c]], ztsem[p], add=True)
    pltpu.async_copy(exz_a.at[p], zsh_a.at[dstv.at[c]], zasem[p], add=True)

  def wait_z(c, p):
    pltpu.make_async_copy(exz_t.at[p], zsh_t.at[dstv.at[c]],
                          ztsem[p]).wait()
    pltpu.make_async_copy(exz_a.at[p], zsh_a.at[dstv.at[c]],
                          zasem[p]).wait()

  compute(0, 0)
  fire_z(0, 0)
  compute(1, 1)
  fire_z(1, 1)

  def chunk2(t, carry):
    c0 = 2 * t + 2
    wait_z(c0 - 2, 0)
    compute(c0, 0)
    fire_z(c0, 0)
    wait_z(c0 - 1, 1)
    compute(c0 + 1, 1)
    fire_z(c0 + 1, 1)
    return carry

  lax.fori_loop(0, (NCH - 3) // 2, chunk2, 0)
  wait_z(NCH - 3, 0)
  compute(NCH - 1, 0)
  fire_z(NCH - 1, 0)
  wait_z(NCH - 2, 1)
  wait_z(NCH - 1, 0)

  pltpu.sync_copy(packv, pack_out.at[wid])
  plsc.subcore_barrier()
  pltpu.sync_copy(zsh_t.at[pl.ds(sid * RW, RW)],
                  zt_out.at[cid, pl.ds(sid * RW, RW)])
  pltpu.sync_copy(zsh_a.at[pl.ds(sid * RW, RW)],
                  za_out.at[cid, pl.ds(sid * RW, RW)])


def _make_msgpass(d, segs):
  @functools.partial(
      pl.kernel,
      out_type=jax.ShapeDtypeStruct((NC, NP, d), F32),
      mesh=_mesh,
      scratch_types=[
          [pltpu.VMEM((4, K), jnp.int32)] * 4,
          [pltpu.VMEM((K, d), F32)] * 2,
          pltpu.VMEM((16, d), F32),
          pltpu.VMEM_SHARED((NP, d), F32),
          [pltpu.SemaphoreType.DMA] * 4,
          [pltpu.SemaphoreType.DMA] * 2,
          [pltpu.SemaphoreType.DMA] * 2,
      ],
      compiler_params=pltpu.CompilerParams(needs_layout_passes=False, use_tc_tiling_on_sc=False),
  )
  def msg_kernel(x_hbm, pack_hbm, acc_out,
                 idxb, rowsb, zrow, acc, isem, gsem, ssem):
    cid = lax.axis_index("c")
    sid = lax.axis_index("s")
    wid = sid * NC + cid

    zero16 = jnp.zeros((16,), F32)
    for i in range(16):
      for j in range(d // 16):
        zrow[i, pl.ds(j * 16, 16)] = zero16
    for i in range(RW // 16):
      pltpu.sync_copy(zrow, acc.at[pl.ds(sid * RW + i * 16, 16)])
    plsc.subcore_barrier()

    def start_idx(c, ib):
      pltpu.async_copy(pack_hbm.at[wid, c], idxb[ib], isem[ib])

    def wait_idx(ib):
      pltpu.make_async_copy(pack_hbm.at[wid, 0], idxb[ib], isem[ib]).wait()

    def start_gather(ib, b):
      pltpu.async_copy(x_hbm.at[idxb[ib].at[0]], rowsb[b], gsem[b])

    def wait_gather(ib, b):
      pltpu.make_async_copy(x_hbm.at[idxb[ib].at[0]], rowsb[b], gsem[b]).wait()

    def start_scatter(ib, b):
      pltpu.async_copy(rowsb[b], acc.at[idxb[ib].at[1]], ssem[b], add=True)

    def wait_scatter(ib, b):
      pltpu.make_async_copy(rowsb[b], acc.at[idxb[ib].at[1]], ssem[b]).wait()

    def scale(ib, b):
      rows = rowsb[b]

      def sbody(k0, carry):
        base = k0 * 16
        for kk in range(16):
          for (prow, jlo, jhi) in segs:
            wbits = plsc.load_gather(
                idxb[ib], [jnp.full((16,), prow, jnp.int32),
                           jnp.full((16,), base + kk, jnp.int32)])
            w = plsc.bitcast(wbits, F32)
            for j in range(jlo, jhi):
              rows[base + kk, pl.ds(j * 16, 16)] = (
                  rows[base + kk, pl.ds(j * 16, 16)] * w)
        return carry

      lax.fori_loop(0, K // 16, sbody, 0)

    def step(c_dyn, cm, first, has_next, has_next2):
      ib, b = cm % 4, cm % 2
      wait_gather(ib, b)
      scale(ib, b)
      start_scatter(ib, b)
      if not first:
        wait_scatter((ib - 1) % 4, b ^ 1)
      if has_next:
        wait_idx((ib + 1) % 4)
        start_gather((ib + 1) % 4, b ^ 1)
      if has_next2:
        start_idx(c_dyn + 2, (ib + 2) % 4)

    start_idx(0, 0)
    start_idx(1, 1)
    wait_idx(0)
    start_gather(0, 0)
    step(0, 0, True, True, True)
    step(1, 1, False, True, True)
    step(2, 2, False, True, True)

    def body(t, carry):
      c0 = 4 * t + 3
      step(c0, 3, False, True, True)
      step(c0 + 1, 0, False, True, True)
      step(c0 + 2, 1, False, True, True)
      step(c0 + 3, 2, False, True, True)
      return carry

    lax.fori_loop(0, (NCH - 5) // 4, body, 0)
    step(NCH - 2, (NCH - 2) % 4, False, True, False)
    step(NCH - 1, (NCH - 1) % 4, False, False, False)
    wait_scatter((NCH - 1) % 4, (NCH - 1) % 2)

    plsc.subcore_barrier()
    pltpu.sync_copy(acc.at[pl.ds(sid * RW, RW)],
                    acc_out.at[cid, pl.ds(sid * RW, RW)])

  return msg_kernel


_msgpass_t = _make_msgpass(128, ((2, 0, 8),))
_msgpass_a = _make_msgpass(128, ((3, 0, 8),))
_msgpass_cat = _make_msgpass(128, ((2, 0, 4), (3, 4, 8)))


def _layer1(acc, z3, w2):
  h1, h2 = w2.shape

  def body(acc_ref, z_ref, w2_ref, x2_ref):
    den = z_ref[0] + z_ref[1] + 1e-16
    g = jnp.maximum((acc_ref[0] + acc_ref[1]) / den, 0.0)
    x2_ref[...] = _hi_dot(g, w2_ref[...])

  return pl.pallas_call(
      body,
      grid=(NB,),
      in_specs=[
          pl.BlockSpec((NC, BR, h1), lambda i: (0, i, 0)),
          pl.BlockSpec((NC, BR, 1), lambda i: (0, i, 0)),
          pl.BlockSpec((h1, h2), lambda i: (0, 0)),
      ],
      out_specs=pl.BlockSpec((BR, h2), lambda i: (i, 0)),
      out_shape=jax.ShapeDtypeStruct((N, h2), F32),
  )(acc, z3, w2)


def _final(acc2_t, acc2_a, z3_t, z3_a, h_t, h_a, wc1, bc1, wc2, bc2):
  h2 = acc2_t.shape[2]
  ut = h_t.shape[1]
  ua = h_a.shape[1]
  hc = wc1.shape[1]
  tags = wc2.shape[1]
  w_g2t = wc1[0:h2]
  w_g2a = wc1[h2:2 * h2]
  w_ha = wc1[2 * h2:2 * h2 + ua]
  w_ht = wc1[2 * h2 + ua:]
  bc1r = bc1.reshape(1, hc)
  bc2r = bc2.reshape(1, tags)

  def body(a2t_ref, a2a_ref, zt_ref, za_ref, ht_ref, ha_ref,
           wg2t_ref, wg2a_ref, wha_ref, wht_ref, b1_ref, wc2_ref, b2_ref,
           out_ref):
    g2t = (a2t_ref[0] + a2t_ref[1]) / (zt_ref[0] + zt_ref[1] + 1e-16)
    g2a = (a2a_ref[0] + a2a_ref[1]) / (za_ref[0] + za_ref[1] + 1e-16)
    hid = (_hi_dot(g2t, wg2t_ref[...]) + _hi_dot(g2a, wg2a_ref[...])
           + _hi_dot(ha_ref[...], wha_ref[...])
           + _hi_dot(ht_ref[...], wht_ref[...]) + b1_ref[...])
    hid = jnp.maximum(hid, 0.0)
    lg = _hi_dot(hid, wc2_ref[...]) + b2_ref[...]
    m = jnp.max(lg, axis=1, keepdims=True)
    p = lg - m
    out_ref[...] = p - jnp.log(jnp.sum(jnp.exp(p), axis=1, keepdims=True))

  return pl.pallas_call(
      body,
      grid=(NB,),
      in_specs=[
          pl.BlockSpec((NC, BR, h2), lambda i: (0, i, 0)),
          pl.BlockSpec((NC, BR, h2), lambda i: (0, i, 0)),
          pl.BlockSpec((NC, BR, 1), lambda i: (0, i, 0)),
          pl.BlockSpec((NC, BR, 1), lambda i: (0, i, 0)),
          pl.BlockSpec((BR, ut), lambda i: (i, 0)),
          pl.BlockSpec((BR, ua), lambda i: (i, 0)),
          pl.BlockSpec((h2, hc), lambda i: (0, 0)),
          pl.BlockSpec((h2, hc), lambda i: (0, 0)),
          pl.BlockSpec((ua, hc), lambda i: (0, 0)),
          pl.BlockSpec((ut, hc), lambda i: (0, 0)),
          pl.BlockSpec((1, hc), lambda i: (0, 0)),
          pl.BlockSpec((hc, tags), lambda i: (0, 0)),
          pl.BlockSpec((1, tags), lambda i: (0, 0)),
      ],
      out_specs=pl.BlockSpec((BR, tags), lambda i: (i, 0)),
      out_shape=jax.ShapeDtypeStruct((N, tags), F32),
  )(acc2_t, acc2_a, z3_t, z3_a, h_t, h_a,
    w_g2t, w_g2a, w_ha, w_ht, bc1r, wc2, bc2r)


def kernel(train_text, train_audio, edge_index, W_rnn_t, b_rnn_t, W_rnn_a,
           b_rnn_a, a_src_t, a_dst_t, a_src_a, a_dst_a, W1_t, W2_t, W1_a,
           W2_a, Wc1, bc1, Wc2, bc2):
  src3 = edge_index[0].reshape(NW, NCH, K)
  dst3 = edge_index[1].reshape(NW, NCH, K)

  h_t, x1_t, sd_t = _encode(train_text, W_rnn_t, b_rnn_t,
                            a_src_t, a_dst_t, W1_t)
  h_a, x1_a, sd_a = _encode(train_audio, W_rnn_a, b_rnn_a,
                            a_src_a, a_dst_a, W1_a)

  pack, z_t, z_a = _attn_kernel(sd_t, sd_a, src3, dst3)
  z3_t = z_t.reshape(NC, NP, 1)
  z3_a = z_a.reshape(NC, NP, 1)

  acc1_t = _msgpass_t(x1_t, pack)
  acc1_a = _msgpass_a(x1_a, pack)

  x2_t = _layer1(acc1_t, z3_t, W2_t)
  x2_a = _layer1(acc1_a, z3_a, W2_a)

  x2_cat = jnp.concatenate([x2_t, x2_a], axis=1)
  acc2 = _msgpass_cat(x2_cat, pack)
  acc2_t = acc2[:, :, 0:64]
  acc2_a = acc2[:, :, 64:128]

  return _final(acc2_t, acc2_a, z3_t, z3_a, h_t, h_a, Wc1, bc1, Wc2, bc2)
